# baseline (speedup 1.0000x reference)
import functools

import jax
import jax.numpy as jnp
from jax import lax
from jax.experimental import pallas as pl
from jax.experimental.pallas import tpu as pltpu
from jax.experimental.pallas import tpu_sc as plsc

DIM = 64
B = 1024
L = 200

_info = plsc.get_sparse_core_info()
NC, NS = _info.num_cores, _info.num_subcores
NW = NC * NS
ROWS_W = B // NW
NBUF = 4


@functools.partial(
    pl.kernel,
    out_type=jax.ShapeDtypeStruct((B, L, DIM), jnp.float32),
    mesh=plsc.VectorSubcoreMesh(core_axis_name="c", subcore_axis_name="s"),
    compiler_params=pltpu.CompilerParams(use_tc_tiling_on_sc=False),
    scratch_types=[
        pltpu.VMEM((ROWS_W * L,), jnp.int32),
        pltpu.VMEM((NBUF, L, DIM), jnp.float32),
        pltpu.SemaphoreType.DMA,
        pltpu.SemaphoreType.DMA,
        pltpu.SemaphoreType.DMA,
    ],
)
def _gather_kernel(table_hbm, idx_hbm, out_hbm, idx_v, rows_v, isem, gsem,
                   wsem):
    wid = lax.axis_index("s") * NC + lax.axis_index("c")
    rbase = wid * ROWS_W

    for m in range(ROWS_W):
        pltpu.async_copy(idx_hbm.at[rbase + m], idx_v.at[pl.ds(m * L, L)],
                         isem)
    for m in range(ROWS_W):
        pltpu.make_async_copy(idx_hbm.at[rbase], idx_v.at[pl.ds(0, L)],
                              isem).wait()

    for b in range(NBUF):
        pltpu.async_copy(
            table_hbm.at[idx_v.at[pl.ds(b * L, L)]], rows_v.at[b], gsem
        )

    @pl.loop(0, ROWS_W)
    def _(j):
        slot = lax.rem(j, NBUF)
        pltpu.make_async_copy(
            table_hbm.at[idx_v.at[pl.ds(0, L)]], rows_v.at[slot], gsem
        ).wait()
        write = pltpu.async_copy(rows_v.at[slot], out_hbm.at[rbase + j], wsem)

        @pl.when(j + NBUF < ROWS_W)
        def _():
            write.wait()
            pltpu.async_copy(
                table_hbm.at[idx_v.at[pl.ds((j + NBUF) * L, L)]],
                rows_v.at[slot],
                gsem,
            )

    for b in range(NBUF):
        pltpu.make_async_copy(rows_v.at[b], out_hbm.at[rbase], wsem).wait()


def kernel(mode, word_inputs, word_seq_lengths, char_inputs, char_seq_lengths,
           char_seq_recover, word_embedding_weight):
    idx = word_inputs.astype(jnp.int32)
    return _gather_kernel(word_embedding_weight, idx)


# speedup vs baseline: 1.0016x; 1.0016x over previous
import functools

import jax
import jax.numpy as jnp
from jax import lax
from jax.experimental import pallas as pl
from jax.experimental.pallas import tpu as pltpu
from jax.experimental.pallas import tpu_sc as plsc

DIM = 64
B = 1024
L = 200

_info = plsc.get_sparse_core_info()
NC, NS = _info.num_cores, _info.num_subcores
NW = NC * NS
ROWS_W = B // NW
NBUF = 4


@functools.partial(
    pl.kernel,
    out_type=jax.ShapeDtypeStruct((B, L, DIM), jnp.float32),
    mesh=plsc.VectorSubcoreMesh(core_axis_name="c", subcore_axis_name="s"),
    compiler_params=pltpu.CompilerParams(use_tc_tiling_on_sc=False,
                                         skip_device_barrier=True),
    scratch_types=[
        pltpu.VMEM((ROWS_W * L,), jnp.int32),
        pltpu.VMEM((NBUF, L, DIM), jnp.float32),
        pltpu.SemaphoreType.DMA,
        pltpu.SemaphoreType.DMA,
        pltpu.SemaphoreType.DMA,
    ],
)
def _gather_kernel(table_hbm, idx_hbm, out_hbm, idx_v, rows_v, isem, gsem,
                   wsem):
    wid = lax.axis_index("s") * NC + lax.axis_index("c")
    rbase = wid * ROWS_W

    for m in range(ROWS_W):
        pltpu.async_copy(idx_hbm.at[rbase + m], idx_v.at[pl.ds(m * L, L)],
                         isem)
    for m in range(ROWS_W):
        pltpu.make_async_copy(idx_hbm.at[rbase], idx_v.at[pl.ds(0, L)],
                              isem).wait()

    for b in range(NBUF):
        pltpu.async_copy(
            table_hbm.at[idx_v.at[pl.ds(b * L, L)]], rows_v.at[b], gsem
        )

    @pl.loop(0, ROWS_W)
    def _(j):
        slot = lax.rem(j, NBUF)
        pltpu.make_async_copy(
            table_hbm.at[idx_v.at[pl.ds(0, L)]], rows_v.at[slot], gsem
        ).wait()
        write = pltpu.async_copy(rows_v.at[slot], out_hbm.at[rbase + j], wsem)

        @pl.when(j + NBUF < ROWS_W)
        def _():
            write.wait()
            pltpu.async_copy(
                table_hbm.at[idx_v.at[pl.ds((j + NBUF) * L, L)]],
                rows_v.at[slot],
                gsem,
            )

    for b in range(NBUF):
        pltpu.make_async_copy(rows_v.at[b], out_hbm.at[rbase], wsem).wait()


def kernel(mode, word_inputs, word_seq_lengths, char_inputs, char_seq_lengths,
           char_seq_recover, word_embedding_weight):
    idx = word_inputs.astype(jnp.int32)
    return _gather_kernel(word_embedding_weight, idx)
